# Initial kernel scaffold; baseline (speedup 1.0000x reference)
import jax
import jax.numpy as jnp
from jax.experimental import pallas as pl


def kernel(x, edge_index, W1, b1, W2, b2, Wfc, bfc):
    raise NotImplementedError("write your pallas kernel here")


# speedup vs baseline: 5.5766x; 5.5766x over previous
import functools

import jax
import jax.numpy as jnp
from jax import lax
from jax.experimental import pallas as pl
from jax.experimental.pallas import tpu as pltpu
from jax.experimental.pallas import tpu_sc as plsc

N = 10000
E = 320000
NC, NS = 2, 16
C = 128
EPT = 20480
EPAD = NS * EPT - E
CHUNKS = EPT // C
PAD_DST = 10239
HALF = 5120
TRASH = HALF
HPAD = HALF + 8
HSTRIPE = HALF // NS
NPAD = 10240
DROWS = NPAD // 128
RBLK = 2048

_mesh = plsc.VectorSubcoreMesh(core_axis_name="c", subcore_axis_name="s")


@functools.partial(
    pl.kernel,
    out_type=jax.ShapeDtypeStruct((NC, NS, DROWS, 128), jnp.float32),
    mesh=_mesh,
    scratch_types=[
        pltpu.VMEM((CHUNKS, C), jnp.int32),
        pltpu.VMEM((DROWS, 128), jnp.float32),
    ],
    compiler_params=pltpu.CompilerParams(needs_layout_passes=False),
)
def _deg_kernel(dst_hbm, zeros_hbm, out_hbm, dst_v, deg_v):
    c = lax.axis_index("c")
    s = lax.axis_index("s")
    pltpu.sync_copy(dst_hbm.at[s], dst_v)
    pltpu.sync_copy(zeros_hbm.at[pl.ds(0, DROWS)], deg_v)
    ones = jnp.ones((16,), jnp.float32)

    def body(j, carry):
        for k in range(C // 16):
            idx = dst_v[j, pl.ds(k * 16, 16)]
            hi = lax.shift_right_logical(idx, 7)
            lo = lax.bitwise_and(idx, 127)
            plsc.addupdate_scatter(deg_v, [hi, lo], ones)
        return carry

    lax.fori_loop(c * (CHUNKS // NC), (c + 1) * (CHUNKS // NC), body, 0)
    pltpu.sync_copy(deg_v, out_hbm.at[c, s])


@functools.partial(
    pl.kernel,
    out_type=jax.ShapeDtypeStruct((NC, HALF, 128), jnp.float32),
    mesh=_mesh,
    scratch_types=[
        pltpu.VMEM((CHUNKS, C), jnp.int32),
        pltpu.VMEM((CHUNKS, C), jnp.int32),
        pltpu.VMEM((C, 128), jnp.float32),
        pltpu.VMEM_SHARED((HPAD, 128), jnp.float32),
        pltpu.SemaphoreType.DMA,
    ],
    compiler_params=pltpu.CompilerParams(needs_layout_passes=False),
)
def _agg(h_hbm, src_hbm, dst_hbm, zeros_hbm, out_hbm,
         src_v, dst_v, rows_v, acc_sh, sem):
    c = lax.axis_index("c")
    s = lax.axis_index("s")

    ICH = 16

    def icopy(j, carry):
        pltpu.sync_copy(src_hbm.at[s, pl.ds(j * ICH, ICH)],
                        src_v.at[pl.ds(j * ICH, ICH)])
        pltpu.sync_copy(dst_hbm.at[s, pl.ds(j * ICH, ICH)],
                        dst_v.at[pl.ds(j * ICH, ICH)])
        return carry

    lax.fori_loop(0, CHUNKS // ICH, icopy, 0)
    pltpu.sync_copy(zeros_hbm, acc_sh.at[pl.ds(s * HSTRIPE, HSTRIPE)])

    @pl.when(s == 0)
    def _():
        pltpu.sync_copy(zeros_hbm.at[pl.ds(0, 8)], acc_sh.at[pl.ds(HALF, 8)])

    base = c * HALF

    def tbody(j, carry):
        for k in range(C // 16):
            d = dst_v[j, pl.ds(k * 16, 16)] - base
            ok = (d >= 0) & (d < HALF)
            dst_v[j, pl.ds(k * 16, 16)] = jnp.where(ok, d, TRASH)
        return carry

    lax.fori_loop(0, CHUNKS, tbody, 0)
    plsc.subcore_barrier()

    def body(j, carry):
        pltpu.async_copy(h_hbm.at[src_v.at[j]], rows_v, sem).wait()
        pltpu.sync_copy(rows_v, acc_sh.at[dst_v.at[j]], add=True)
        return carry

    lax.fori_loop(0, CHUNKS, body, 0)
    plsc.subcore_barrier()
    pltpu.sync_copy(acc_sh.at[pl.ds(s * HSTRIPE, HSTRIPE)],
                    out_hbm.at[c, pl.ds(s * HSTRIPE, HSTRIPE)])


def _dinv_of(deg_ref):
    deg = jnp.sum(deg_ref[...], axis=(0, 1))[:, None] + 1.0
    return lax.rsqrt(deg)


def _tc1_body(x_ref, w_ref, deg_ref, out_ref):
    dinv = _dinv_of(deg_ref)
    out_ref[...] = jnp.dot(x_ref[...], w_ref[...],
                           preferred_element_type=jnp.float32) * dinv


def _tc2_body(acc_ref, h_ref, deg_ref, b_ref, w_ref, out_ref):
    dinv = _dinv_of(deg_ref)
    z = (acc_ref[...] + h_ref[...]) * dinv + b_ref[...]
    z = jnp.maximum(z, 0.0)
    h2 = jnp.dot(z, w_ref[...], preferred_element_type=jnp.float32) * dinv
    out_ref[...] = jnp.concatenate([h2, jnp.zeros_like(h2)], axis=1)


def _tc3_body(acc_ref, h_ref, deg_ref, b_ref, w_ref, bfc_ref, out_ref):
    dinv = _dinv_of(deg_ref)
    z = (acc_ref[:, :64] + h_ref[:, :64]) * dinv + b_ref[...]
    z = jnp.maximum(z, 0.0)
    out_ref[...] = jnp.dot(z, w_ref[...],
                           preferred_element_type=jnp.float32) + bfc_ref[...]


def _tc1(x, W1, deg):
    return pl.pallas_call(
        _tc1_body,
        grid=(pl.cdiv(N, RBLK),),
        in_specs=[
            pl.BlockSpec((RBLK, 128), lambda i: (i, 0)),
            pl.BlockSpec((128, 128), lambda i: (0, 0)),
            pl.BlockSpec((NC, NS, RBLK), lambda i: (0, 0, i)),
        ],
        out_specs=pl.BlockSpec((RBLK, 128), lambda i: (i, 0)),
        out_shape=jax.ShapeDtypeStruct((N, 128), jnp.float32),
    )(x, W1, deg)


def _tc2(acc, h1p, deg, b1, W2):
    return pl.pallas_call(
        _tc2_body,
        grid=(pl.cdiv(N, RBLK),),
        in_specs=[
            pl.BlockSpec((RBLK, 128), lambda i: (i, 0)),
            pl.BlockSpec((RBLK, 128), lambda i: (i, 0)),
            pl.BlockSpec((NC, NS, RBLK), lambda i: (0, 0, i)),
            pl.BlockSpec((1, 128), lambda i: (0, 0)),
            pl.BlockSpec((128, 64), lambda i: (0, 0)),
        ],
        out_specs=pl.BlockSpec((RBLK, 128), lambda i: (i, 0)),
        out_shape=jax.ShapeDtypeStruct((N, 128), jnp.float32),
    )(acc, h1p, deg, b1, W2)


def _tc3(acc, h2p, deg, b2, Wfc, bfc):
    return pl.pallas_call(
        _tc3_body,
        grid=(pl.cdiv(N, RBLK),),
        in_specs=[
            pl.BlockSpec((RBLK, 128), lambda i: (i, 0)),
            pl.BlockSpec((RBLK, 128), lambda i: (i, 0)),
            pl.BlockSpec((NC, NS, RBLK), lambda i: (0, 0, i)),
            pl.BlockSpec((1, 64), lambda i: (0, 0)),
            pl.BlockSpec((64, 1), lambda i: (0, 0)),
            pl.BlockSpec((1, 1), lambda i: (0, 0)),
        ],
        out_specs=pl.BlockSpec((RBLK, 1), lambda i: (i, 0)),
        out_shape=jax.ShapeDtypeStruct((N, 1), jnp.float32),
    )(acc, h2p, deg, b2, Wfc, bfc)


def kernel(x, edge_index, W1, b1, W2, b2, Wfc, bfc):
    src = jnp.concatenate(
        [edge_index[0].astype(jnp.int32),
         jnp.zeros((EPAD,), jnp.int32)]).reshape(NS, CHUNKS, C)
    dst = jnp.concatenate(
        [edge_index[1].astype(jnp.int32),
         jnp.full((EPAD,), PAD_DST, jnp.int32)]).reshape(NS, CHUNKS, C)
    zeros128 = jnp.zeros((HSTRIPE, 128), jnp.float32)

    deg = _deg_kernel(dst, zeros128)
    deg = deg.reshape(NC, NS, NPAD)
    h1p = _tc1(x, W1, deg)
    acc1 = _agg(h1p, src, dst, zeros128)
    acc1 = acc1.reshape(NPAD, 128)[:N]
    h2p = _tc2(acc1, h1p, deg, b1.reshape(1, -1), W2)
    acc2 = _agg(h2p, src, dst, zeros128)
    acc2 = acc2.reshape(NPAD, 128)[:N]
    out = _tc3(acc2, h2p, deg, b2.reshape(1, -1), Wfc, bfc.reshape(1, 1))
    return out


# speedup vs baseline: 5.7635x; 1.0335x over previous
import functools

import jax
import jax.numpy as jnp
from jax import lax
from jax.experimental import pallas as pl
from jax.experimental.pallas import tpu as pltpu
from jax.experimental.pallas import tpu_sc as plsc

N = 10000
E = 320000
NC, NS = 2, 16
C = 128
CHUNKS = 160
EPT = CHUNKS * C
EPAD = NS * EPT - E
PAD_DST = 10239
B0 = 5104
TRASH = B0
ACC_ROWS = 5112
NPAD = 10240
DROWS = NPAD // 128
RBLK = 2048

_mesh = plsc.VectorSubcoreMesh(core_axis_name="c", subcore_axis_name="s")


@functools.partial(
    pl.kernel,
    out_type=jax.ShapeDtypeStruct((NC, NS, DROWS, 128), jnp.float32),
    mesh=_mesh,
    scratch_types=[
        pltpu.VMEM((CHUNKS, C), jnp.int32),
        pltpu.VMEM((DROWS, 128), jnp.float32),
    ],
    compiler_params=pltpu.CompilerParams(needs_layout_passes=False),
)
def _deg_kernel(dst_hbm, zeros_hbm, out_hbm, dst_v, deg_v):
    c = lax.axis_index("c")
    s = lax.axis_index("s")
    pltpu.sync_copy(dst_hbm.at[s], dst_v)
    pltpu.sync_copy(zeros_hbm.at[pl.ds(0, DROWS)], deg_v)
    ones = jnp.ones((16,), jnp.float32)

    def body(j, carry):
        for k in range(C // 16):
            idx = dst_v[j, pl.ds(k * 16, 16)]
            hi = lax.shift_right_logical(idx, 7)
            lo = lax.bitwise_and(idx, 127)
            plsc.addupdate_scatter(deg_v, [hi, lo], ones)
        return carry

    lax.fori_loop(c * (CHUNKS // NC), (c + 1) * (CHUNKS // NC), body, 0)
    pltpu.sync_copy(deg_v, out_hbm.at[c, s])


@functools.partial(
    pl.kernel,
    out_type=jax.ShapeDtypeStruct((NC, ACC_ROWS, 128), jnp.float32),
    mesh=_mesh,
    scratch_types=[
        pltpu.VMEM((CHUNKS, C), jnp.int32),
        pltpu.VMEM((CHUNKS, C), jnp.int32),
        pltpu.VMEM((C, 128), jnp.float32),
        pltpu.VMEM((C, 128), jnp.float32),
        pltpu.VMEM((C, 128), jnp.float32),
        pltpu.VMEM_SHARED((ACC_ROWS, 128), jnp.float32),
        pltpu.SemaphoreType.DMA,
        pltpu.SemaphoreType.DMA,
        pltpu.SemaphoreType.DMA,
    ],
    compiler_params=pltpu.CompilerParams(needs_layout_passes=False),
)
def _agg(h_hbm, src_hbm, dst_hbm, zeros_hbm, out_hbm,
         src_v, dst_v, rows0, rows1, rows2, acc_sh, sem0, sem1, sem2):
    c = lax.axis_index("c")
    s = lax.axis_index("s")

    ICH = 8

    def icopy(j, carry):
        pltpu.sync_copy(src_hbm.at[s, pl.ds(j * ICH, ICH)],
                        src_v.at[pl.ds(j * ICH, ICH)])
        pltpu.sync_copy(dst_hbm.at[s, pl.ds(j * ICH, ICH)],
                        dst_v.at[pl.ds(j * ICH, ICH)])
        return carry

    lax.fori_loop(0, CHUNKS // ICH, icopy, 0)

    nz = jnp.where(s == NS - 1, 7, 8)

    def zcopy(k, carry):
        pltpu.sync_copy(zeros_hbm, acc_sh.at[pl.ds(s * 320 + k * 40, 40)])
        return carry

    lax.fori_loop(0, nz, zcopy, 0)

    @pl.when(s == NS - 1)
    def _():
        pltpu.sync_copy(zeros_hbm.at[pl.ds(0, 32)], acc_sh.at[pl.ds(5080, 32)])

    base = c * B0

    def tbody(j, carry):
        for k in range(C // 16):
            d = dst_v[j, pl.ds(k * 16, 16)] - base
            ok = (d >= 0) & (d < B0)
            dst_v[j, pl.ds(k * 16, 16)] = jnp.where(ok, d, TRASH)
        return carry

    lax.fori_loop(0, CHUNKS, tbody, 0)
    plsc.subcore_barrier()

    bufs = (rows0, rows1, rows2)
    sems = (sem0, sem1, sem2)
    NBUF = 3
    for b in range(NBUF):
        pltpu.async_copy(h_hbm.at[src_v.at[b]], bufs[b], sems[b])

    def body(i, carry):
        for b in range(NBUF):
            j = i * NBUF + b

            @pl.when(j < CHUNKS)
            def _():
                pltpu.make_async_copy(h_hbm.at[src_v.at[j]],
                                      bufs[b], sems[b]).wait()
                pltpu.sync_copy(bufs[b], acc_sh.at[dst_v.at[j]], add=True)

                @pl.when(j + NBUF < CHUNKS)
                def _():
                    pltpu.async_copy(h_hbm.at[src_v.at[j + NBUF]],
                                     bufs[b], sems[b])
        return carry

    lax.fori_loop(0, pl.cdiv(CHUNKS, NBUF), body, 0)
    plsc.subcore_barrier()

    no = jnp.where(s == NS - 1, 7, 8)

    def ocopy(k, carry):
        pltpu.sync_copy(acc_sh.at[pl.ds(s * 320 + k * 40, 40)],
                        out_hbm.at[c, pl.ds(s * 320 + k * 40, 40)])
        return carry

    lax.fori_loop(0, no, ocopy, 0)

    @pl.when(s == NS - 1)
    def _():
        pltpu.sync_copy(acc_sh.at[pl.ds(5080, 32)],
                        out_hbm.at[c, pl.ds(5080, 32)])


def _dinv_of(deg_ref):
    deg = jnp.sum(deg_ref[...], axis=(0, 1))[:, None] + 1.0
    return lax.rsqrt(deg)


def _tc1_body(x_ref, w_ref, deg_ref, out_ref):
    dinv = _dinv_of(deg_ref)
    out_ref[...] = jnp.dot(x_ref[...], w_ref[...],
                           preferred_element_type=jnp.float32) * dinv


def _tc2_body(acc_ref, h_ref, deg_ref, b_ref, w_ref, out_ref):
    dinv = _dinv_of(deg_ref)
    z = (acc_ref[...] + h_ref[...]) * dinv + b_ref[...]
    z = jnp.maximum(z, 0.0)
    h2 = jnp.dot(z, w_ref[...], preferred_element_type=jnp.float32) * dinv
    out_ref[...] = jnp.concatenate([h2, jnp.zeros_like(h2)], axis=1)


def _tc3_body(acc_ref, h_ref, deg_ref, b_ref, w_ref, bfc_ref, out_ref):
    dinv = _dinv_of(deg_ref)
    z = (acc_ref[:, :64] + h_ref[:, :64]) * dinv + b_ref[...]
    z = jnp.maximum(z, 0.0)
    out_ref[...] = jnp.dot(z, w_ref[...],
                           preferred_element_type=jnp.float32) + bfc_ref[...]


def _tc1(x, W1, deg):
    return pl.pallas_call(
        _tc1_body,
        grid=(pl.cdiv(N, RBLK),),
        in_specs=[
            pl.BlockSpec((RBLK, 128), lambda i: (i, 0)),
            pl.BlockSpec((128, 128), lambda i: (0, 0)),
            pl.BlockSpec((NC, NS, RBLK), lambda i: (0, 0, i)),
        ],
        out_specs=pl.BlockSpec((RBLK, 128), lambda i: (i, 0)),
        out_shape=jax.ShapeDtypeStruct((N, 128), jnp.float32),
    )(x, W1, deg)


def _tc2(acc, h1p, deg, b1, W2):
    return pl.pallas_call(
        _tc2_body,
        grid=(pl.cdiv(N, RBLK),),
        in_specs=[
            pl.BlockSpec((RBLK, 128), lambda i: (i, 0)),
            pl.BlockSpec((RBLK, 128), lambda i: (i, 0)),
            pl.BlockSpec((NC, NS, RBLK), lambda i: (0, 0, i)),
            pl.BlockSpec((1, 128), lambda i: (0, 0)),
            pl.BlockSpec((128, 64), lambda i: (0, 0)),
        ],
        out_specs=pl.BlockSpec((RBLK, 128), lambda i: (i, 0)),
        out_shape=jax.ShapeDtypeStruct((N, 128), jnp.float32),
    )(acc, h1p, deg, b1, W2)


def _tc3(acc, h2p, deg, b2, Wfc, bfc):
    return pl.pallas_call(
        _tc3_body,
        grid=(pl.cdiv(N, RBLK),),
        in_specs=[
            pl.BlockSpec((RBLK, 128), lambda i: (i, 0)),
            pl.BlockSpec((RBLK, 128), lambda i: (i, 0)),
            pl.BlockSpec((NC, NS, RBLK), lambda i: (0, 0, i)),
            pl.BlockSpec((1, 64), lambda i: (0, 0)),
            pl.BlockSpec((64, 1), lambda i: (0, 0)),
            pl.BlockSpec((1, 1), lambda i: (0, 0)),
        ],
        out_specs=pl.BlockSpec((RBLK, 1), lambda i: (i, 0)),
        out_shape=jax.ShapeDtypeStruct((N, 1), jnp.float32),
    )(acc, h2p, deg, b2, Wfc, bfc)


def kernel(x, edge_index, W1, b1, W2, b2, Wfc, bfc):
    src = jnp.concatenate(
        [edge_index[0].astype(jnp.int32),
         jnp.zeros((EPAD,), jnp.int32)]).reshape(NS, CHUNKS, C)
    dst = jnp.concatenate(
        [edge_index[1].astype(jnp.int32),
         jnp.full((EPAD,), PAD_DST, jnp.int32)]).reshape(NS, CHUNKS, C)
    zeros128 = jnp.zeros((40, 128), jnp.float32)

    deg = _deg_kernel(dst, zeros128)
    deg = deg.reshape(NC, NS, NPAD)
    h1p = _tc1(x, W1, deg)
    acc1 = _agg(h1p, src, dst, zeros128)
    acc1 = jnp.concatenate([acc1[0, :B0], acc1[1, :N - B0]], axis=0)
    h2p = _tc2(acc1, h1p, deg, b1.reshape(1, -1), W2)
    acc2 = _agg(h2p, src, dst, zeros128)
    acc2 = jnp.concatenate([acc2[0, :B0], acc2[1, :N - B0]], axis=0)
    out = _tc3(acc2, h2p, deg, b2.reshape(1, -1), Wfc, bfc.reshape(1, 1))
    return out


# speedup vs baseline: 5.9127x; 1.0259x over previous
import functools

import jax
import jax.numpy as jnp
from jax import lax
from jax.experimental import pallas as pl
from jax.experimental.pallas import tpu as pltpu
from jax.experimental.pallas import tpu_sc as plsc

N = 10000
E = 320000
NC, NS = 2, 16
C = 128
EPT = 20480
EPAD = NS * EPT - E
CHUNKS = EPT // C
PAD_DST = 10239
HALF = 5120
TRASH = HALF
HPAD = HALF + 8
HSTRIPE = HALF // NS
NPAD = 10240
DROWS = NPAD // 128
RBLK = 2048

_mesh = plsc.VectorSubcoreMesh(core_axis_name="c", subcore_axis_name="s")


@functools.partial(
    pl.kernel,
    out_type=jax.ShapeDtypeStruct((NC, NS, DROWS, 128), jnp.float32),
    mesh=_mesh,
    scratch_types=[
        pltpu.VMEM((CHUNKS, C), jnp.int32),
        pltpu.VMEM((DROWS, 128), jnp.float32),
    ],
    compiler_params=pltpu.CompilerParams(needs_layout_passes=False),
)
def _deg_kernel(dst_hbm, zeros_hbm, out_hbm, dst_v, deg_v):
    c = lax.axis_index("c")
    s = lax.axis_index("s")
    pltpu.sync_copy(dst_hbm.at[s], dst_v)
    pltpu.sync_copy(zeros_hbm.at[pl.ds(0, DROWS)], deg_v)
    ones = jnp.ones((16,), jnp.float32)

    def body(j, carry):
        for k in range(C // 16):
            idx = dst_v[j, pl.ds(k * 16, 16)]
            hi = lax.shift_right_logical(idx, 7)
            lo = lax.bitwise_and(idx, 127)
            plsc.addupdate_scatter(deg_v, [hi, lo], ones)
        return carry

    lax.fori_loop(c * (CHUNKS // NC), (c + 1) * (CHUNKS // NC), body, 0)
    pltpu.sync_copy(deg_v, out_hbm.at[c, s])


@functools.partial(
    pl.kernel,
    out_type=jax.ShapeDtypeStruct((NC, HALF, 128), jnp.float32),
    mesh=_mesh,
    scratch_types=[
        pltpu.VMEM((CHUNKS, C), jnp.int32),
        pltpu.VMEM((CHUNKS, C), jnp.int32),
        pltpu.VMEM((C, 128), jnp.float32),
        pltpu.VMEM((C, 128), jnp.float32),
        pltpu.VMEM_SHARED((HPAD, 128), jnp.float32),
        pltpu.SemaphoreType.DMA,
        pltpu.SemaphoreType.DMA,
    ],
    compiler_params=pltpu.CompilerParams(needs_layout_passes=False),
)
def _agg(h_hbm, src_hbm, dst_hbm, zeros_hbm, out_hbm,
         src_v, dst_v, rows0, rows1, acc_sh, sem0, sem1):
    c = lax.axis_index("c")
    s = lax.axis_index("s")

    ICH = 16

    def icopy(j, carry):
        pltpu.sync_copy(src_hbm.at[s, pl.ds(j * ICH, ICH)],
                        src_v.at[pl.ds(j * ICH, ICH)])
        pltpu.sync_copy(dst_hbm.at[s, pl.ds(j * ICH, ICH)],
                        dst_v.at[pl.ds(j * ICH, ICH)])
        return carry

    lax.fori_loop(0, CHUNKS // ICH, icopy, 0)
    pltpu.sync_copy(zeros_hbm, acc_sh.at[pl.ds(s * HSTRIPE, HSTRIPE)])

    @pl.when(s == 0)
    def _():
        pltpu.sync_copy(zeros_hbm.at[pl.ds(0, 8)], acc_sh.at[pl.ds(HALF, 8)])

    base = c * HALF

    def tbody(j, carry):
        for k in range(C // 16):
            d = dst_v[j, pl.ds(k * 16, 16)] - base
            ok = (d >= 0) & (d < HALF)
            dst_v[j, pl.ds(k * 16, 16)] = jnp.where(ok, d, TRASH)
        return carry

    lax.fori_loop(0, CHUNKS, tbody, 0)
    plsc.subcore_barrier()

    bufs = (rows0, rows1)
    sems = (sem0, sem1)
    NBUF = 2
    for b in range(NBUF):
        pltpu.async_copy(h_hbm.at[src_v.at[b]], bufs[b], sems[b])

    def body(i, carry):
        for b in range(NBUF):
            j = i * NBUF + b
            pltpu.make_async_copy(h_hbm.at[src_v.at[j]], bufs[b], sems[b]).wait()
            pltpu.sync_copy(bufs[b], acc_sh.at[dst_v.at[j]], add=True)

            @pl.when(i < CHUNKS // NBUF - 1)
            def _():
                pltpu.async_copy(h_hbm.at[src_v.at[j + NBUF]], bufs[b], sems[b])
        return carry

    lax.fori_loop(0, CHUNKS // NBUF, body, 0)
    plsc.subcore_barrier()
    pltpu.sync_copy(acc_sh.at[pl.ds(s * HSTRIPE, HSTRIPE)],
                    out_hbm.at[c, pl.ds(s * HSTRIPE, HSTRIPE)])


def _dinv_of(deg_ref):
    deg = jnp.sum(deg_ref[...], axis=(0, 1))[:, None] + 1.0
    return lax.rsqrt(deg)


def _tc1_body(x_ref, w_ref, deg_ref, out_ref):
    dinv = _dinv_of(deg_ref)
    out_ref[...] = jnp.dot(x_ref[...], w_ref[...],
                           preferred_element_type=jnp.float32) * dinv


def _tc2_body(acc_ref, h_ref, deg_ref, b_ref, w_ref, out_ref):
    dinv = _dinv_of(deg_ref)
    z = (acc_ref[...] + h_ref[...]) * dinv + b_ref[...]
    z = jnp.maximum(z, 0.0)
    h2 = jnp.dot(z, w_ref[...], preferred_element_type=jnp.float32) * dinv
    out_ref[...] = jnp.concatenate([h2, jnp.zeros_like(h2)], axis=1)


def _tc3_body(acc_ref, h_ref, deg_ref, b_ref, w_ref, bfc_ref, out_ref):
    dinv = _dinv_of(deg_ref)
    z = (acc_ref[:, :64] + h_ref[:, :64]) * dinv + b_ref[...]
    z = jnp.maximum(z, 0.0)
    out_ref[...] = jnp.dot(z, w_ref[...],
                           preferred_element_type=jnp.float32) + bfc_ref[...]


def _tc1(x, W1, deg):
    return pl.pallas_call(
        _tc1_body,
        grid=(pl.cdiv(N, RBLK),),
        in_specs=[
            pl.BlockSpec((RBLK, 128), lambda i: (i, 0)),
            pl.BlockSpec((128, 128), lambda i: (0, 0)),
            pl.BlockSpec((NC, NS, RBLK), lambda i: (0, 0, i)),
        ],
        out_specs=pl.BlockSpec((RBLK, 128), lambda i: (i, 0)),
        out_shape=jax.ShapeDtypeStruct((N, 128), jnp.float32),
    )(x, W1, deg)


def _tc2(acc, h1p, deg, b1, W2):
    return pl.pallas_call(
        _tc2_body,
        grid=(pl.cdiv(N, RBLK),),
        in_specs=[
            pl.BlockSpec((RBLK, 128), lambda i: (i, 0)),
            pl.BlockSpec((RBLK, 128), lambda i: (i, 0)),
            pl.BlockSpec((NC, NS, RBLK), lambda i: (0, 0, i)),
            pl.BlockSpec((1, 128), lambda i: (0, 0)),
            pl.BlockSpec((128, 64), lambda i: (0, 0)),
        ],
        out_specs=pl.BlockSpec((RBLK, 128), lambda i: (i, 0)),
        out_shape=jax.ShapeDtypeStruct((N, 128), jnp.float32),
    )(acc, h1p, deg, b1, W2)


def _tc3(acc, h2p, deg, b2, Wfc, bfc):
    return pl.pallas_call(
        _tc3_body,
        grid=(pl.cdiv(N, RBLK),),
        in_specs=[
            pl.BlockSpec((RBLK, 128), lambda i: (i, 0)),
            pl.BlockSpec((RBLK, 128), lambda i: (i, 0)),
            pl.BlockSpec((NC, NS, RBLK), lambda i: (0, 0, i)),
            pl.BlockSpec((1, 64), lambda i: (0, 0)),
            pl.BlockSpec((64, 1), lambda i: (0, 0)),
            pl.BlockSpec((1, 1), lambda i: (0, 0)),
        ],
        out_specs=pl.BlockSpec((RBLK, 1), lambda i: (i, 0)),
        out_shape=jax.ShapeDtypeStruct((N, 1), jnp.float32),
    )(acc, h2p, deg, b2, Wfc, bfc)


def kernel(x, edge_index, W1, b1, W2, b2, Wfc, bfc):
    src = jnp.concatenate(
        [edge_index[0].astype(jnp.int32),
         jnp.zeros((EPAD,), jnp.int32)]).reshape(NS, CHUNKS, C)
    dst = jnp.concatenate(
        [edge_index[1].astype(jnp.int32),
         jnp.full((EPAD,), PAD_DST, jnp.int32)]).reshape(NS, CHUNKS, C)
    zeros128 = jnp.zeros((HSTRIPE, 128), jnp.float32)

    deg = _deg_kernel(dst, zeros128)
    deg = deg.reshape(NC, NS, NPAD)
    h1p = _tc1(x, W1, deg)
    acc1 = _agg(h1p, src, dst, zeros128)
    acc1 = acc1.reshape(NPAD, 128)[:N]
    h2p = _tc2(acc1, h1p, deg, b1.reshape(1, -1), W2)
    acc2 = _agg(h2p, src, dst, zeros128)
    acc2 = acc2.reshape(NPAD, 128)[:N]
    out = _tc3(acc2, h2p, deg, b2.reshape(1, -1), Wfc, bfc.reshape(1, 1))
    return out
